# baseline (speedup 1.0000x reference)
import functools

import jax
import jax.numpy as jnp
from jax import lax
from jax.experimental import pallas as pl
from jax.experimental.pallas import tpu as pltpu
from jax.experimental.pallas import tpu_sc as plsc

B, S, G = 1024, 50, 8
D = 64
N = B * S * G
BS = B * S
NC, NS = 2, 16
NW = NC * NS
PER_W = N // NW
ROWS_W = BS // NW
CHUNK = 128
SLAB = 1280
NSLAB = PER_W // SLAB
CH_PER_SLAB = SLAB // CHUNK
OUT_SLAB = SLAB // G
LANES = 16

_mesh = plsc.VectorSubcoreMesh(core_axis_name="c", subcore_axis_name="s")


@functools.partial(
    pl.kernel,
    mesh=_mesh,
    compiler_params=pltpu.CompilerParams(use_tc_tiling_on_sc=False),
    out_type=jax.ShapeDtypeStruct((BS, D), jnp.float32),
    scratch_types=[
        pltpu.VMEM((PER_W,), jnp.int32),
        pltpu.VMEM((PER_W,), jnp.float32),
        pltpu.VMEM((ROWS_W,), jnp.float32),
        pltpu.VMEM((CHUNK, D), jnp.float32),
        pltpu.VMEM((OUT_SLAB, D), jnp.float32),
        pltpu.SemaphoreType.DMA,
    ],
)
def _gaz_embed_sc(idx_hbm, mask_hbm, len_hbm, table_hbm, out_hbm,
                  idx_v, mask_v, len_v, rows_v, out_v, sem):
    wid = lax.axis_index("s") * NC + lax.axis_index("c")
    ibase = wid * PER_W
    rbase = wid * ROWS_W
    pltpu.sync_copy(idx_hbm.at[pl.ds(ibase, PER_W)], idx_v)
    pltpu.sync_copy(mask_hbm.at[pl.ds(ibase, PER_W)], mask_v)
    pltpu.sync_copy(len_hbm.at[pl.ds(rbase, ROWS_W)], len_v)

    def slab_body(s_i, _):
        soff = s_i * SLAB

        def chunk_body(c_i, _):
            coff = soff + c_i * CHUNK
            pltpu.async_copy(
                table_hbm.at[idx_v.at[pl.ds(coff, CHUNK)]], rows_v, sem
            ).wait()
            obase = c_i * (CHUNK // G)
            inv_vec = 1.0 / len_v[pl.ds(s_i * OUT_SLAB + obase, LANES)]
            for half in range(CHUNK // LANES):
                mv = mask_v[pl.ds(coff + half * LANES, LANES)]
                for sub in range(2):
                    r = half * 2 + sub
                    r0 = r * G
                    inv = inv_vec[r]
                    for d_blk in range(D // LANES):
                        dsl = pl.ds(d_blk * LANES, LANES)
                        acc = rows_v[r0, dsl] * mv[sub * G]
                        for g in range(1, G):
                            acc = acc + rows_v[r0 + g, dsl] * mv[sub * G + g]
                        out_v[obase + r, dsl] = acc * inv
            return 0

        lax.fori_loop(0, CH_PER_SLAB, chunk_body, 0)
        pltpu.sync_copy(
            out_v, out_hbm.at[pl.ds(rbase + s_i * OUT_SLAB, OUT_SLAB)]
        )
        return 0

    lax.fori_loop(0, NSLAB, slab_body, 0)


def kernel(gaz_seq_tensor, gaz_seq_lengths, gaz_mask_tensor, gaz_embedding):
    idx = gaz_seq_tensor.reshape(N).astype(jnp.int32)
    mask = gaz_mask_tensor.reshape(N)
    lens = gaz_seq_lengths.reshape(BS).astype(jnp.float32)
    out = _gaz_embed_sc(idx, mask, lens, gaz_embedding)
    return out.reshape(B, S, D)
